# Initial kernel scaffold; baseline (speedup 1.0000x reference)
import jax
import jax.numpy as jnp
from jax.experimental import pallas as pl


def kernel(x, emb, W1, b1, W2, b2, W3, b3):
    raise NotImplementedError("write your pallas kernel here")


# speedup vs baseline: 29.5330x; 29.5330x over previous
import functools

import jax
import jax.numpy as jnp
from jax import lax
from jax.experimental import pallas as pl
from jax.experimental.pallas import tpu as pltpu
from jax.experimental.pallas import tpu_sc as plsc

VOCAB = 14923
D = 64
H = 128
C = 10
B = 16384
L = 200

NC, NS = 2, 16
NW = NC * NS
ROWS_PER_W = B // NW
RB = 64
N_CHUNKS = ROWS_PER_W // RB
NB = 4

_LANES = 16
_DV = D // _LANES


def _pool_body(x_hbm, emb_hbm, out_hbm, idxs_v, r0_v, r1_v, r2_v, r3_v,
               out_v, s0, s1, s2, s3):
    rows = (r0_v, r1_v, r2_v, r3_v)
    sems = (s0, s1, s2, s3)
    wid = lax.axis_index("s") * NC + lax.axis_index("c")
    base = wid * ROWS_PER_W

    def chunk_body(c, _):
        row0 = base + c * RB
        pltpu.sync_copy(x_hbm.at[pl.ds(row0 * L, RB * L)], idxs_v)
        for b in range(NB):
            pltpu.async_copy(
                emb_hbm.at[idxs_v.at[pl.ds(b * L, L)]], rows[b], sems[b]
            )

        def group_body(g, _):
            for b in range(NB):
                r = g * NB + b
                pltpu.make_async_copy(
                    emb_hbm.at[idxs_v.at[pl.ds(0, L)]], rows[b], sems[b]
                ).wait()

                def tok_body(t, accs, b=b):
                    return tuple(
                        a + rows[b][t, pl.ds(j * _LANES, _LANES)]
                        for j, a in enumerate(accs)
                    )

                accs = lax.fori_loop(
                    0, L, tok_body,
                    tuple(jnp.zeros((_LANES,), jnp.float32)
                          for _ in range(_DV)),
                )
                inv_l = jnp.float32(1.0 / L)
                for j in range(_DV):
                    out_v[r, pl.ds(j * _LANES, _LANES)] = accs[j] * inv_l

                nxt = r + NB

                @pl.when(nxt < RB)
                def _(b=b, nxt=nxt):
                    pltpu.async_copy(
                        emb_hbm.at[idxs_v.at[pl.ds(nxt * L, L)]],
                        rows[b], sems[b],
                    )

            return ()

        lax.fori_loop(0, RB // NB, group_body, ())
        pltpu.sync_copy(out_v, out_hbm.at[pl.ds(row0, RB)])
        return ()

    lax.fori_loop(0, N_CHUNKS, chunk_body, ())


_pool = functools.partial(
    pl.kernel,
    mesh=plsc.VectorSubcoreMesh(core_axis_name="c", subcore_axis_name="s"),
    out_type=jax.ShapeDtypeStruct((B, D), jnp.float32),
    compiler_params=pltpu.CompilerParams(use_tc_tiling_on_sc=False),
    scratch_types=[
        pltpu.VMEM((RB * L,), jnp.int32),
        pltpu.VMEM((L, D), jnp.float32),
        pltpu.VMEM((L, D), jnp.float32),
        pltpu.VMEM((L, D), jnp.float32),
        pltpu.VMEM((L, D), jnp.float32),
        pltpu.VMEM((RB, D), jnp.float32),
        pltpu.SemaphoreType.DMA,
        pltpu.SemaphoreType.DMA,
        pltpu.SemaphoreType.DMA,
        pltpu.SemaphoreType.DMA,
    ],
)(_pool_body)


def _mlp_body(h_ref, w1_ref, b1_ref, w2_ref, b2_ref, w3_ref, b3_ref, out_ref):
    h = h_ref[:]
    z1 = jnp.maximum(
        jnp.dot(h, w1_ref[:], preferred_element_type=jnp.float32) + b1_ref[:], 0.0
    )
    z2 = jnp.maximum(
        jnp.dot(z1, w2_ref[:], preferred_element_type=jnp.float32) + b2_ref[:], 0.0
    )
    logits = jnp.dot(z2, w3_ref[:], preferred_element_type=jnp.float32) + b3_ref[:]
    m = jnp.max(logits, axis=-1, keepdims=True)
    e = jnp.exp(logits - m)
    s = jnp.sum(e, axis=-1, keepdims=True)
    out_ref[:] = logits - m - jnp.log(s)


_MLP_BLOCK = 2048


def _mlp(h, W1, b1, W2, b2, W3, b3):
    grid = (B // _MLP_BLOCK,)
    full = lambda shape: pl.BlockSpec(shape, lambda i: (0, 0))
    return pl.pallas_call(
        _mlp_body,
        grid=grid,
        in_specs=[
            pl.BlockSpec((_MLP_BLOCK, D), lambda i: (i, 0)),
            full((D, H)),
            full((1, H)),
            full((H, H)),
            full((1, H)),
            full((H, C)),
            full((1, C)),
        ],
        out_specs=pl.BlockSpec((_MLP_BLOCK, C), lambda i: (i, 0)),
        out_shape=jax.ShapeDtypeStruct((B, C), jnp.float32),
    )(h, W1, b1, W2, b2, W3, b3)


def kernel(x, emb, W1, b1, W2, b2, W3, b3):
    xf = x.astype(jnp.int32).reshape(-1)
    h = _pool(xf, emb)
    return _mlp(
        h,
        W1, b1.reshape(1, H),
        W2, b2.reshape(1, H),
        W3, b3.reshape(1, C),
    )


# speedup vs baseline: 45.7609x; 1.5495x over previous
import functools

import jax
import jax.numpy as jnp
import numpy as np
from jax import lax
from jax.experimental import pallas as pl
from jax.experimental.pallas import tpu as pltpu
from jax.experimental.pallas import tpu_sc as plsc

VOCAB = 14923
D = 64
H = 128
C = 10
B = 16384
L = 200

NC, NS = 2, 16
NW = NC * NS
ROWS_PER_W = B // NW
RB = 64
N_CHUNKS = ROWS_PER_W // RB
NB = 4

_LANES = 16
_DV = D // _LANES
_TU = 4

_PERM = np.concatenate([
    np.arange(0, 32, 2), np.arange(1, 32, 2),
    np.arange(32, 64, 2), np.arange(33, 64, 2),
])


def _pool_body(x_hbm, emb_hbm, out_hbm, idxs_v, r0_v, r1_v, r2_v, r3_v,
               out_v, s0, s1, s2, s3):
    rows = (r0_v, r1_v, r2_v, r3_v)
    sems = (s0, s1, s2, s3)
    wid = lax.axis_index("s") * NC + lax.axis_index("c")
    base = wid * ROWS_PER_W

    def chunk_body(c, _):
        row0 = base + c * RB
        pltpu.sync_copy(x_hbm.at[pl.ds(row0 * L, RB * L)], idxs_v)
        for b in range(NB):
            pltpu.async_copy(
                emb_hbm.at[idxs_v.at[pl.ds(b * L, L)]], rows[b], sems[b]
            )

        def group_body(g, _):
            for b in range(NB):
                r = g * NB + b
                pltpu.make_async_copy(
                    emb_hbm.at[idxs_v.at[pl.ds(0, L)]], rows[b], sems[b]
                ).wait()

                def tok_body(ti, accs, b=b):
                    accs = list(accs)
                    for u in range(_TU):
                        t = ti * _TU + u
                        for g in range(2):
                            v = rows[b][t, pl.ds(g * 32, 32)]
                            ea, eb = plsc.unpack(
                                v, format=plsc.PackFormat.INTERLEAVED
                            )
                            accs[2 * g] = accs[2 * g] + ea
                            accs[2 * g + 1] = accs[2 * g + 1] + eb
                    return tuple(accs)

                accs = lax.fori_loop(
                    0, L // _TU, tok_body,
                    tuple(jnp.zeros((_LANES,), jnp.float32)
                          for _ in range(_DV)),
                )
                inv_l = jnp.float32(1.0 / L)
                for j in range(_DV):
                    out_v[r, pl.ds(j * _LANES, _LANES)] = accs[j] * inv_l

                nxt = r + NB

                @pl.when(nxt < RB)
                def _(b=b, nxt=nxt):
                    pltpu.async_copy(
                        emb_hbm.at[idxs_v.at[pl.ds(nxt * L, L)]],
                        rows[b], sems[b],
                    )

            return ()

        lax.fori_loop(0, RB // NB, group_body, ())
        pltpu.sync_copy(out_v, out_hbm.at[pl.ds(row0, RB)])
        return ()

    lax.fori_loop(0, N_CHUNKS, chunk_body, ())


_pool = functools.partial(
    pl.kernel,
    mesh=plsc.VectorSubcoreMesh(core_axis_name="c", subcore_axis_name="s"),
    out_type=jax.ShapeDtypeStruct((B, D), jnp.float32),
    compiler_params=pltpu.CompilerParams(
        use_tc_tiling_on_sc=False, needs_layout_passes=False
    ),
    scratch_types=[
        pltpu.VMEM((RB * L,), jnp.int32),
        pltpu.VMEM((L, D), jnp.bfloat16),
        pltpu.VMEM((L, D), jnp.bfloat16),
        pltpu.VMEM((L, D), jnp.bfloat16),
        pltpu.VMEM((L, D), jnp.bfloat16),
        pltpu.VMEM((RB, D), jnp.float32),
        pltpu.SemaphoreType.DMA,
        pltpu.SemaphoreType.DMA,
        pltpu.SemaphoreType.DMA,
        pltpu.SemaphoreType.DMA,
    ],
)(_pool_body)


def _mlp_body(h_ref, w1_ref, b1_ref, w2_ref, b2_ref, w3_ref, b3_ref, out_ref):
    h = h_ref[:]
    z1 = jnp.maximum(
        jnp.dot(h, w1_ref[:], preferred_element_type=jnp.float32) + b1_ref[:], 0.0
    )
    z2 = jnp.maximum(
        jnp.dot(z1, w2_ref[:], preferred_element_type=jnp.float32) + b2_ref[:], 0.0
    )
    logits = jnp.dot(z2, w3_ref[:], preferred_element_type=jnp.float32) + b3_ref[:]
    m = jnp.max(logits, axis=-1, keepdims=True)
    e = jnp.exp(logits - m)
    s = jnp.sum(e, axis=-1, keepdims=True)
    out_ref[:] = logits - m - jnp.log(s)


_MLP_BLOCK = 2048


def _mlp(h, W1, b1, W2, b2, W3, b3):
    grid = (B // _MLP_BLOCK,)
    full = lambda shape: pl.BlockSpec(shape, lambda i: (0, 0))
    return pl.pallas_call(
        _mlp_body,
        grid=grid,
        in_specs=[
            pl.BlockSpec((_MLP_BLOCK, D), lambda i: (i, 0)),
            full((D, H)),
            full((1, H)),
            full((H, H)),
            full((1, H)),
            full((H, C)),
            full((1, C)),
        ],
        out_specs=pl.BlockSpec((_MLP_BLOCK, C), lambda i: (i, 0)),
        out_shape=jax.ShapeDtypeStruct((B, C), jnp.float32),
    )(h, W1, b1, W2, b2, W3, b3)


def kernel(x, emb, W1, b1, W2, b2, W3, b3):
    xf = x.astype(jnp.int32).reshape(-1)
    h = _pool(xf, emb.astype(jnp.bfloat16))
    return _mlp(
        h,
        W1[jnp.asarray(_PERM)], b1.reshape(1, H),
        W2, b2.reshape(1, H),
        W3, b3.reshape(1, C),
    )


# speedup vs baseline: 65.5407x; 1.4322x over previous
import functools

import jax
import jax.numpy as jnp
from jax import lax
from jax.experimental import pallas as pl
from jax.experimental.pallas import tpu as pltpu
from jax.experimental.pallas import tpu_sc as plsc

VOCAB = 14923
D = 64
H = 128
C = 10
B = 16384
L = 200

NC, NS = 2, 16
NW = NC * NS
ROWS_PER_W = B // NW
RB = 64
N_CHUNKS = ROWS_PER_W // RB
NB = 4

_LANES = 16
_DV = D // _LANES
_TG = 20
_SCALE = 64.0

_VSLICE = 933
_VCHUNK = 311
VOCAB_PAD = 16 * _VSLICE


def _pool_body(x_hbm, emb_hbm, out_hbm, idxs_v, r0_v, r1_v, r2_v, r3_v,
               out_v, stage_v, conv_v, spm, s0, s1, s2, s3):
    rows = (r0_v, r1_v, r2_v, r3_v)
    sems = (s0, s1, s2, s3)
    sid = lax.axis_index("s")
    wid = sid * NC + lax.axis_index("c")
    base = wid * ROWS_PER_W

    scale = jnp.float32(_SCALE)
    for c in range(3):
        src0 = sid * _VSLICE + c * _VCHUNK

        if c < 2:
            pltpu.sync_copy(emb_hbm.at[pl.ds(src0, _VCHUNK)], stage_v)
        else:
            @pl.when(sid < NS - 1)
            def _(src0=src0):
                pltpu.sync_copy(emb_hbm.at[pl.ds(src0, _VCHUNK)], stage_v)

            @pl.when(sid == NS - 1)
            def _(src0=src0):
                pltpu.sync_copy(
                    emb_hbm.at[pl.ds(src0, _VCHUNK - 5)],
                    stage_v.at[pl.ds(0, _VCHUNK - 5)],
                )

        def conv_body(i, _):
            w = stage_v[i, pl.ds(0, 16)] * scale
            x_ = stage_v[i, pl.ds(16, 16)] * scale
            y = stage_v[i, pl.ds(32, 16)] * scale
            z = stage_v[i, pl.ds(48, 16)] * scale
            a = plsc.pack(w, x_, format=plsc.PackFormat.INTERLEAVED)
            bvec = plsc.pack(y, z, format=plsc.PackFormat.INTERLEAVED)
            conv_v[i, pl.ds(0, 64)] = plsc.pack(
                a, bvec, format=plsc.PackFormat.INTERLEAVED,
                preferred_element_type=jnp.float8_e4m3fn,
            )
            return ()

        lax.fori_loop(0, _VCHUNK, conv_body, ())
        pltpu.sync_copy(conv_v, spm.at[pl.ds(src0, _VCHUNK)])

    plsc.subcore_barrier()

    def chunk_body(c, _):
        row0 = base + c * RB
        pltpu.sync_copy(x_hbm.at[pl.ds(row0 * L, RB * L)], idxs_v)
        for b in range(NB):
            pltpu.async_copy(
                spm.at[idxs_v.at[pl.ds(b * L, L)]], rows[b], sems[b]
            )

        def group_body(g, _):
            for b in range(NB):
                r = g * NB + b
                pltpu.make_async_copy(
                    spm.at[idxs_v.at[pl.ds(0, L)]], rows[b], sems[b]
                ).wait()

                def tok_body(ti, accs, b=b):
                    acc_a = jnp.zeros((32,), jnp.bfloat16)
                    acc_b = jnp.zeros((32,), jnp.bfloat16)
                    for u in range(_TG):
                        t = ti * _TG + u
                        v = rows[b][t, pl.ds(0, 64)]
                        ea, eb = plsc.unpack(
                            v, format=plsc.PackFormat.INTERLEAVED,
                            preferred_element_type=jnp.bfloat16,
                        )
                        acc_a = acc_a + ea
                        acc_b = acc_b + eb
                    aa, ab = plsc.unpack(
                        acc_a, format=plsc.PackFormat.INTERLEAVED
                    )
                    ba, bb = plsc.unpack(
                        acc_b, format=plsc.PackFormat.INTERLEAVED
                    )
                    return (accs[0] + aa, accs[1] + ab,
                            accs[2] + ba, accs[3] + bb)

                accs = lax.fori_loop(
                    0, L // _TG, tok_body,
                    tuple(jnp.zeros((_LANES,), jnp.float32)
                          for _ in range(_DV)),
                )
                inv_l = jnp.float32(1.0 / (L * _SCALE))
                for j in range(_DV):
                    out_v[r, pl.ds(j * _LANES, _LANES)] = accs[j] * inv_l

                nxt = r + NB

                @pl.when(nxt < RB)
                def _(b=b, nxt=nxt):
                    pltpu.async_copy(
                        spm.at[idxs_v.at[pl.ds(nxt * L, L)]],
                        rows[b], sems[b],
                    )

            return ()

        lax.fori_loop(0, RB // NB, group_body, ())
        pltpu.sync_copy(out_v, out_hbm.at[pl.ds(row0, RB)])
        return ()

    lax.fori_loop(0, N_CHUNKS, chunk_body, ())


_pool = functools.partial(
    pl.kernel,
    mesh=plsc.VectorSubcoreMesh(core_axis_name="c", subcore_axis_name="s"),
    out_type=jax.ShapeDtypeStruct((B, D), jnp.float32),
    compiler_params=pltpu.CompilerParams(
        use_tc_tiling_on_sc=False, needs_layout_passes=False
    ),
    scratch_types=[
        pltpu.VMEM((RB * L,), jnp.int32),
        pltpu.VMEM((L, D), jnp.float8_e4m3fn),
        pltpu.VMEM((L, D), jnp.float8_e4m3fn),
        pltpu.VMEM((L, D), jnp.float8_e4m3fn),
        pltpu.VMEM((L, D), jnp.float8_e4m3fn),
        pltpu.VMEM((RB, D), jnp.float32),
        pltpu.VMEM((_VCHUNK, D), jnp.float32),
        pltpu.VMEM((_VCHUNK, D), jnp.float8_e4m3fn),
        pltpu.VMEM_SHARED((VOCAB_PAD, D), jnp.float8_e4m3fn),
        pltpu.SemaphoreType.DMA,
        pltpu.SemaphoreType.DMA,
        pltpu.SemaphoreType.DMA,
        pltpu.SemaphoreType.DMA,
    ],
)(_pool_body)


def _mlp_body(h_ref, w1_ref, b1_ref, w2_ref, b2_ref, w3_ref, b3_ref, out_ref):
    h = h_ref[:]
    z1 = jnp.maximum(
        jnp.dot(h, w1_ref[:], preferred_element_type=jnp.float32) + b1_ref[:], 0.0
    )
    z2 = jnp.maximum(
        jnp.dot(z1, w2_ref[:], preferred_element_type=jnp.float32) + b2_ref[:], 0.0
    )
    logits = jnp.dot(z2, w3_ref[:], preferred_element_type=jnp.float32) + b3_ref[:]
    m = jnp.max(logits, axis=-1, keepdims=True)
    e = jnp.exp(logits - m)
    s = jnp.sum(e, axis=-1, keepdims=True)
    out_ref[:] = logits - m - jnp.log(s)


_MLP_BLOCK = 2048


def _mlp(h, W1, b1, W2, b2, W3, b3):
    grid = (B // _MLP_BLOCK,)
    full = lambda shape: pl.BlockSpec(shape, lambda i: (0, 0))
    return pl.pallas_call(
        _mlp_body,
        grid=grid,
        in_specs=[
            pl.BlockSpec((_MLP_BLOCK, D), lambda i: (i, 0)),
            full((D, H)),
            full((1, H)),
            full((H, H)),
            full((1, H)),
            full((H, C)),
            full((1, C)),
        ],
        out_specs=pl.BlockSpec((_MLP_BLOCK, C), lambda i: (i, 0)),
        out_shape=jax.ShapeDtypeStruct((B, C), jnp.float32),
    )(h, W1, b1, W2, b2, W3, b3)


def kernel(x, emb, W1, b1, W2, b2, W3, b3):
    xf = x.astype(jnp.int32).reshape(-1)
    h = _pool(xf, emb)
    return _mlp(
        h,
        W1, b1.reshape(1, H),
        W2, b2.reshape(1, H),
        W3, b3.reshape(1, C),
    )


# speedup vs baseline: 65.7522x; 1.0032x over previous
import functools

import jax
import jax.numpy as jnp
from jax import lax
from jax.experimental import pallas as pl
from jax.experimental.pallas import tpu as pltpu
from jax.experimental.pallas import tpu_sc as plsc

VOCAB = 14923
D = 64
H = 128
C = 10
B = 16384
L = 200

NC, NS = 2, 16
NW = NC * NS
ROWS_PER_W = B // NW
RB = 64
N_CHUNKS = ROWS_PER_W // RB
NB = 4

_LANES = 16
_DV = D // _LANES
_TG = 20
_SCALE = 64.0

_VSLICE = 933
_VCHUNK = 311
VOCAB_PAD = 16 * _VSLICE


def _pool_body(x_hbm, emb_hbm, out_hbm, idxs_v, r0_v, r1_v, r2_v, r3_v,
               out_v, stage_v, conv_v, spm, s0, s1, s2, s3):
    rows = (r0_v, r1_v, r2_v, r3_v)
    sems = (s0, s1, s2, s3)
    sid = lax.axis_index("s")
    wid = sid * NC + lax.axis_index("c")
    base = wid * ROWS_PER_W

    scale = jnp.float32(_SCALE)
    for c in range(3):
        src0 = sid * _VSLICE + c * _VCHUNK

        if c < 2:
            pltpu.sync_copy(emb_hbm.at[pl.ds(src0, _VCHUNK)], stage_v)
        else:
            @pl.when(sid < NS - 1)
            def _(src0=src0):
                pltpu.sync_copy(emb_hbm.at[pl.ds(src0, _VCHUNK)], stage_v)

            @pl.when(sid == NS - 1)
            def _(src0=src0):
                pltpu.sync_copy(
                    emb_hbm.at[pl.ds(src0, _VCHUNK - 5)],
                    stage_v.at[pl.ds(0, _VCHUNK - 5)],
                )

        def conv_body(i, _):
            w = stage_v[i, pl.ds(0, 16)] * scale
            x_ = stage_v[i, pl.ds(16, 16)] * scale
            y = stage_v[i, pl.ds(32, 16)] * scale
            z = stage_v[i, pl.ds(48, 16)] * scale
            a = plsc.pack(w, x_, format=plsc.PackFormat.INTERLEAVED)
            bvec = plsc.pack(y, z, format=plsc.PackFormat.INTERLEAVED)
            conv_v[i, pl.ds(0, 64)] = plsc.pack(
                a, bvec, format=plsc.PackFormat.INTERLEAVED,
                preferred_element_type=jnp.float8_e4m3fn,
            )
            return ()

        lax.fori_loop(0, _VCHUNK, conv_body, ())
        pltpu.sync_copy(conv_v, spm.at[pl.ds(src0, _VCHUNK)])

    plsc.subcore_barrier()

    def chunk_body(c, _):
        row0 = base + c * RB
        pltpu.sync_copy(x_hbm.at[pl.ds(row0, RB)], idxs_v)
        for b in range(NB):
            pltpu.async_copy(
                spm.at[idxs_v.at[b]], rows[b], sems[b]
            )

        def group_body(g, _):
            for b in range(NB):
                r = g * NB + b
                pltpu.make_async_copy(
                    spm.at[idxs_v.at[0]], rows[b], sems[b]
                ).wait()

                def tok_body(ti, accs, b=b):
                    acc_a = jnp.zeros((32,), jnp.bfloat16)
                    acc_b = jnp.zeros((32,), jnp.bfloat16)
                    for u in range(_TG):
                        t = ti * _TG + u
                        v = rows[b][t, pl.ds(0, 64)]
                        ea, eb = plsc.unpack(
                            v, format=plsc.PackFormat.INTERLEAVED,
                            preferred_element_type=jnp.bfloat16,
                        )
                        acc_a = acc_a + ea
                        acc_b = acc_b + eb
                    aa, ab = plsc.unpack(
                        acc_a, format=plsc.PackFormat.INTERLEAVED
                    )
                    ba, bb = plsc.unpack(
                        acc_b, format=plsc.PackFormat.INTERLEAVED
                    )
                    return (accs[0] + aa, accs[1] + ab,
                            accs[2] + ba, accs[3] + bb)

                accs = lax.fori_loop(
                    0, L // _TG, tok_body,
                    tuple(jnp.zeros((_LANES,), jnp.float32)
                          for _ in range(_DV)),
                )
                inv_l = jnp.float32(1.0 / (L * _SCALE))
                for j in range(_DV):
                    out_v[r, pl.ds(j * _LANES, _LANES)] = accs[j] * inv_l

                nxt = r + NB

                @pl.when(nxt < RB)
                def _(b=b, nxt=nxt):
                    pltpu.async_copy(
                        spm.at[idxs_v.at[nxt]], rows[b], sems[b]
                    )

            return ()

        lax.fori_loop(0, RB // NB, group_body, ())
        pltpu.sync_copy(out_v, out_hbm.at[pl.ds(row0, RB)])
        return ()

    lax.fori_loop(0, N_CHUNKS, chunk_body, ())


_pool = functools.partial(
    pl.kernel,
    mesh=plsc.VectorSubcoreMesh(core_axis_name="c", subcore_axis_name="s"),
    out_type=jax.ShapeDtypeStruct((B, D), jnp.float32),
    compiler_params=pltpu.CompilerParams(
        use_tc_tiling_on_sc=False, needs_layout_passes=False
    ),
    scratch_types=[
        pltpu.VMEM((RB, L), jnp.int32),
        pltpu.VMEM((L, D), jnp.float8_e4m3fn),
        pltpu.VMEM((L, D), jnp.float8_e4m3fn),
        pltpu.VMEM((L, D), jnp.float8_e4m3fn),
        pltpu.VMEM((L, D), jnp.float8_e4m3fn),
        pltpu.VMEM((RB, D), jnp.float32),
        pltpu.VMEM((_VCHUNK, D), jnp.float32),
        pltpu.VMEM((_VCHUNK, D), jnp.float8_e4m3fn),
        pltpu.VMEM_SHARED((VOCAB_PAD, D), jnp.float8_e4m3fn),
        pltpu.SemaphoreType.DMA,
        pltpu.SemaphoreType.DMA,
        pltpu.SemaphoreType.DMA,
        pltpu.SemaphoreType.DMA,
    ],
)(_pool_body)


def _mlp_body(h_ref, w1_ref, b1_ref, w2_ref, b2_ref, w3_ref, b3_ref, out_ref):
    h = h_ref[:]
    z1 = jnp.maximum(
        jnp.dot(h, w1_ref[:], preferred_element_type=jnp.float32) + b1_ref[:], 0.0
    )
    z2 = jnp.maximum(
        jnp.dot(z1, w2_ref[:], preferred_element_type=jnp.float32) + b2_ref[:], 0.0
    )
    logits = jnp.dot(z2, w3_ref[:], preferred_element_type=jnp.float32) + b3_ref[:]
    m = jnp.max(logits, axis=-1, keepdims=True)
    e = jnp.exp(logits - m)
    s = jnp.sum(e, axis=-1, keepdims=True)
    out_ref[:] = logits - m - jnp.log(s)


_MLP_BLOCK = 2048


def _mlp(h, W1, b1, W2, b2, W3, b3):
    grid = (B // _MLP_BLOCK,)
    full = lambda shape: pl.BlockSpec(shape, lambda i: (0, 0))
    return pl.pallas_call(
        _mlp_body,
        grid=grid,
        in_specs=[
            pl.BlockSpec((_MLP_BLOCK, D), lambda i: (i, 0)),
            full((D, H)),
            full((1, H)),
            full((H, H)),
            full((1, H)),
            full((H, C)),
            full((1, C)),
        ],
        out_specs=pl.BlockSpec((_MLP_BLOCK, C), lambda i: (i, 0)),
        out_shape=jax.ShapeDtypeStruct((B, C), jnp.float32),
    )(h, W1, b1, W2, b2, W3, b3)


def kernel(x, emb, W1, b1, W2, b2, W3, b3):
    xf = x.astype(jnp.int32)
    h = _pool(xf, emb)
    return _mlp(
        h,
        W1, b1.reshape(1, H),
        W2, b2.reshape(1, H),
        W3, b3.reshape(1, C),
    )


# speedup vs baseline: 66.5586x; 1.0123x over previous
import functools

import jax
import jax.numpy as jnp
from jax import lax
from jax.experimental import pallas as pl
from jax.experimental.pallas import tpu as pltpu
from jax.experimental.pallas import tpu_sc as plsc

VOCAB = 14923
D = 64
H = 128
C = 10
B = 16384
L = 200

NC, NS = 2, 16
NW = NC * NS
ROWS_PER_W = B // NW
RB = 64
N_CHUNKS = ROWS_PER_W // RB
NB = 4

_LANES = 16
_DV = D // _LANES
_TG = 25
_GR = 2
_SCALE = 64.0

_VSLICE = 933
_VCHUNK = 311
VOCAB_PAD = 16 * _VSLICE


def _pool_body(x_hbm, emb_hbm, out_hbm, idxs_v, r0_v, r1_v, r2_v, r3_v,
               out_v, stage_v, conv_v, spm, s0, s1, s2, s3):
    rows = (r0_v, r1_v, r2_v, r3_v)
    sems = (s0, s1, s2, s3)
    sid = lax.axis_index("s")
    wid = sid * NC + lax.axis_index("c")
    base = wid * ROWS_PER_W

    scale = jnp.float32(_SCALE)
    for c in range(3):
        src0 = sid * _VSLICE + c * _VCHUNK

        if c < 2:
            pltpu.sync_copy(emb_hbm.at[pl.ds(src0, _VCHUNK)], stage_v)
        else:
            @pl.when(sid < NS - 1)
            def _(src0=src0):
                pltpu.sync_copy(emb_hbm.at[pl.ds(src0, _VCHUNK)], stage_v)

            @pl.when(sid == NS - 1)
            def _(src0=src0):
                pltpu.sync_copy(
                    emb_hbm.at[pl.ds(src0, _VCHUNK - 5)],
                    stage_v.at[pl.ds(0, _VCHUNK - 5)],
                )

        def conv_body(i, _):
            w = stage_v[i, pl.ds(0, 16)] * scale
            x_ = stage_v[i, pl.ds(16, 16)] * scale
            y = stage_v[i, pl.ds(32, 16)] * scale
            z = stage_v[i, pl.ds(48, 16)] * scale
            a = plsc.pack(w, x_, format=plsc.PackFormat.INTERLEAVED)
            bvec = plsc.pack(y, z, format=plsc.PackFormat.INTERLEAVED)
            conv_v[i, pl.ds(0, 64)] = plsc.pack(
                a, bvec, format=plsc.PackFormat.INTERLEAVED,
                preferred_element_type=jnp.float8_e4m3fn,
            )
            return ()

        lax.fori_loop(0, _VCHUNK, conv_body, ())
        pltpu.sync_copy(conv_v, spm.at[pl.ds(src0, _VCHUNK)])

    plsc.subcore_barrier()

    inv_l = jnp.float32(1.0 / (L * _SCALE))

    def chunk_body(c, _):
        row0 = base + c * RB
        pltpu.sync_copy(x_hbm.at[pl.ds(row0 * L, RB * L)], idxs_v)
        for b in range(NB):
            pltpu.async_copy(
                spm.at[idxs_v.at[pl.ds(b * _GR * L, _GR * L)]],
                rows[b], sems[b],
            )

        def group_body(g, _):
            for b in range(NB):
                p = g * NB + b
                pltpu.make_async_copy(
                    spm.at[idxs_v.at[pl.ds(0, _GR * L)]], rows[b], sems[b]
                ).wait()

                for rr in range(_GR):

                    def tok_body(ti, accs, b=b, rr=rr):
                        acc_a = jnp.zeros((32,), jnp.bfloat16)
                        acc_b = jnp.zeros((32,), jnp.bfloat16)
                        for u in range(_TG):
                            t = rr * L + ti * _TG + u
                            v = rows[b][t, pl.ds(0, 64)]
                            ea, eb = plsc.unpack(
                                v, format=plsc.PackFormat.INTERLEAVED,
                                preferred_element_type=jnp.bfloat16,
                            )
                            acc_a = acc_a + ea
                            acc_b = acc_b + eb
                        aa, ab = plsc.unpack(
                            acc_a, format=plsc.PackFormat.INTERLEAVED
                        )
                        ba, bb = plsc.unpack(
                            acc_b, format=plsc.PackFormat.INTERLEAVED
                        )
                        return (accs[0] + aa, accs[1] + ab,
                                accs[2] + ba, accs[3] + bb)

                    accs = lax.fori_loop(
                        0, L // _TG, tok_body,
                        tuple(jnp.zeros((_LANES,), jnp.float32)
                              for _ in range(_DV)),
                    )
                    r = p * _GR + rr
                    for j in range(_DV):
                        out_v[r, pl.ds(j * _LANES, _LANES)] = accs[j] * inv_l

                nxt = p + NB

                @pl.when(nxt < RB // _GR)
                def _(b=b, nxt=nxt):
                    pltpu.async_copy(
                        spm.at[idxs_v.at[pl.ds(nxt * _GR * L, _GR * L)]],
                        rows[b], sems[b],
                    )

            return ()

        lax.fori_loop(0, RB // _GR // NB, group_body, ())
        pltpu.sync_copy(out_v, out_hbm.at[pl.ds(row0, RB)])
        return ()

    lax.fori_loop(0, N_CHUNKS, chunk_body, ())


_pool = functools.partial(
    pl.kernel,
    mesh=plsc.VectorSubcoreMesh(core_axis_name="c", subcore_axis_name="s"),
    out_type=jax.ShapeDtypeStruct((B, D), jnp.float32),
    compiler_params=pltpu.CompilerParams(
        use_tc_tiling_on_sc=False, needs_layout_passes=False
    ),
    scratch_types=[
        pltpu.VMEM((RB * L,), jnp.int32),
        pltpu.VMEM((_GR * L, D), jnp.float8_e4m3fn),
        pltpu.VMEM((_GR * L, D), jnp.float8_e4m3fn),
        pltpu.VMEM((_GR * L, D), jnp.float8_e4m3fn),
        pltpu.VMEM((_GR * L, D), jnp.float8_e4m3fn),
        pltpu.VMEM((RB, D), jnp.float32),
        pltpu.VMEM((_VCHUNK, D), jnp.float32),
        pltpu.VMEM((_VCHUNK, D), jnp.float8_e4m3fn),
        pltpu.VMEM_SHARED((VOCAB_PAD, D), jnp.float8_e4m3fn),
        pltpu.SemaphoreType.DMA,
        pltpu.SemaphoreType.DMA,
        pltpu.SemaphoreType.DMA,
        pltpu.SemaphoreType.DMA,
    ],
)(_pool_body)


def _mlp_body(h_ref, w1_ref, b1_ref, w2_ref, b2_ref, w3_ref, b3_ref, out_ref):
    h = h_ref[:]
    z1 = jnp.maximum(
        jnp.dot(h, w1_ref[:], preferred_element_type=jnp.float32) + b1_ref[:], 0.0
    )
    z2 = jnp.maximum(
        jnp.dot(z1, w2_ref[:], preferred_element_type=jnp.float32) + b2_ref[:], 0.0
    )
    logits = jnp.dot(z2, w3_ref[:], preferred_element_type=jnp.float32) + b3_ref[:]
    m = jnp.max(logits, axis=-1, keepdims=True)
    e = jnp.exp(logits - m)
    s = jnp.sum(e, axis=-1, keepdims=True)
    out_ref[:] = logits - m - jnp.log(s)


_MLP_BLOCK = 2048


def _mlp(h, W1, b1, W2, b2, W3, b3):
    grid = (B // _MLP_BLOCK,)
    full = lambda shape: pl.BlockSpec(shape, lambda i: (0, 0))
    return pl.pallas_call(
        _mlp_body,
        grid=grid,
        in_specs=[
            pl.BlockSpec((_MLP_BLOCK, D), lambda i: (i, 0)),
            full((D, H)),
            full((1, H)),
            full((H, H)),
            full((1, H)),
            full((H, C)),
            full((1, C)),
        ],
        out_specs=pl.BlockSpec((_MLP_BLOCK, C), lambda i: (i, 0)),
        out_shape=jax.ShapeDtypeStruct((B, C), jnp.float32),
    )(h, W1, b1, W2, b2, W3, b3)


def kernel(x, emb, W1, b1, W2, b2, W3, b3):
    xf = x.astype(jnp.int32).reshape(-1)
    h = _pool(xf, emb)
    return _mlp(
        h,
        W1, b1.reshape(1, H),
        W2, b2.reshape(1, H),
        W3, b3.reshape(1, C),
    )


# speedup vs baseline: 68.4467x; 1.0284x over previous
import functools

import jax
import jax.numpy as jnp
from jax import lax
from jax.experimental import pallas as pl
from jax.experimental.pallas import tpu as pltpu
from jax.experimental.pallas import tpu_sc as plsc

VOCAB = 14923
D = 64
H = 128
C = 10
B = 16384
L = 200

NC, NS = 2, 16
NW = NC * NS
ROWS_PER_W = B // NW
RB = 64
N_CHUNKS = ROWS_PER_W // RB
NB = 4

_LANES = 16
_DV = D // _LANES
_TG = 25
_GR = 2
_SCALE = 64.0

_VSLICE = 933
_VCHUNK = 311
VOCAB_PAD = 16 * _VSLICE


def _pool_body(x_hbm, emb_hbm, out_hbm, idxs_v, r0_v, r1_v, r2_v, r3_v,
               out_v, stage_v, conv_v, spm, s0, s1, s2, s3):
    rows = (r0_v, r1_v, r2_v, r3_v)
    sems = (s0, s1, s2, s3)
    sid = lax.axis_index("s")
    wid = sid * NC + lax.axis_index("c")
    base = wid * ROWS_PER_W

    scale = jnp.float32(_SCALE)
    for c in range(3):
        src0 = sid * _VSLICE + c * _VCHUNK

        if c < 2:
            pltpu.sync_copy(emb_hbm.at[pl.ds(src0, _VCHUNK)], stage_v)
        else:
            @pl.when(sid < NS - 1)
            def _(src0=src0):
                pltpu.sync_copy(emb_hbm.at[pl.ds(src0, _VCHUNK)], stage_v)

            @pl.when(sid == NS - 1)
            def _(src0=src0):
                pltpu.sync_copy(
                    emb_hbm.at[pl.ds(src0, _VCHUNK - 5)],
                    stage_v.at[pl.ds(0, _VCHUNK - 5)],
                )

        def conv_body(i, _):
            w = stage_v[i, pl.ds(0, 16)] * scale
            x_ = stage_v[i, pl.ds(16, 16)] * scale
            y = stage_v[i, pl.ds(32, 16)] * scale
            z = stage_v[i, pl.ds(48, 16)] * scale
            a = plsc.pack(w, x_, format=plsc.PackFormat.INTERLEAVED)
            bvec = plsc.pack(y, z, format=plsc.PackFormat.INTERLEAVED)
            conv_v[i, pl.ds(0, 64)] = plsc.pack(
                a, bvec, format=plsc.PackFormat.INTERLEAVED,
                preferred_element_type=jnp.float8_e4m3fn,
            )
            return ()

        lax.fori_loop(0, _VCHUNK, conv_body, ())
        pltpu.sync_copy(conv_v, spm.at[pl.ds(src0, _VCHUNK)])

    plsc.subcore_barrier()

    inv_l = jnp.float32(1.0 / (L * _SCALE))

    zero16 = jnp.zeros((_LANES,), jnp.float32)

    def zinit_body(r, _):
        for j in range(4, 8):
            out_v[r // 8, r % 8, pl.ds(j * _LANES, _LANES)] = zero16
        return ()

    lax.fori_loop(0, RB, zinit_body, ())

    def chunk_body(c, _):
        row0 = base + c * RB
        pltpu.sync_copy(x_hbm.at[pl.ds(row0 * L, RB * L)], idxs_v)
        for b in range(NB):
            pltpu.async_copy(
                spm.at[idxs_v.at[pl.ds(b * _GR * L, _GR * L)]],
                rows[b], sems[b],
            )

        def group_body(g, _):
            for b in range(NB):
                p = g * NB + b
                pltpu.make_async_copy(
                    spm.at[idxs_v.at[pl.ds(0, _GR * L)]], rows[b], sems[b]
                ).wait()

                for rr in range(_GR):

                    def tok_body(ti, accs, b=b, rr=rr):
                        acc_a = jnp.zeros((32,), jnp.bfloat16)
                        acc_b = jnp.zeros((32,), jnp.bfloat16)
                        for u in range(_TG):
                            t = rr * L + ti * _TG + u
                            v = rows[b][t, pl.ds(0, 64)]
                            ea, eb = plsc.unpack(
                                v, format=plsc.PackFormat.INTERLEAVED,
                                preferred_element_type=jnp.bfloat16,
                            )
                            acc_a = acc_a + ea
                            acc_b = acc_b + eb
                        aa, ab = plsc.unpack(
                            acc_a, format=plsc.PackFormat.INTERLEAVED
                        )
                        ba, bb = plsc.unpack(
                            acc_b, format=plsc.PackFormat.INTERLEAVED
                        )
                        return (accs[0] + aa, accs[1] + ab,
                                accs[2] + ba, accs[3] + bb)

                    accs = lax.fori_loop(
                        0, L // _TG, tok_body,
                        tuple(jnp.zeros((_LANES,), jnp.float32)
                              for _ in range(_DV)),
                    )
                    r = p * _GR + rr
                    for j in range(_DV):
                        out_v[r // 8, r % 8, pl.ds(j * _LANES, _LANES)] = (
                            accs[j] * inv_l
                        )

                nxt = p + NB

                @pl.when(nxt < RB // _GR)
                def _(b=b, nxt=nxt):
                    pltpu.async_copy(
                        spm.at[idxs_v.at[pl.ds(nxt * _GR * L, _GR * L)]],
                        rows[b], sems[b],
                    )

            return ()

        lax.fori_loop(0, RB // _GR // NB, group_body, ())
        pltpu.sync_copy(out_v, out_hbm.at[pl.ds(row0 // 8, RB // 8)])
        return ()

    lax.fori_loop(0, N_CHUNKS, chunk_body, ())


_pool = functools.partial(
    pl.kernel,
    mesh=plsc.VectorSubcoreMesh(core_axis_name="c", subcore_axis_name="s"),
    out_type=jax.ShapeDtypeStruct((B // 8, 8, 2 * D), jnp.float32),
    compiler_params=pltpu.CompilerParams(
        use_tc_tiling_on_sc=False, needs_layout_passes=False
    ),
    scratch_types=[
        pltpu.VMEM((RB * L,), jnp.int32),
        pltpu.VMEM((_GR * L, D), jnp.float8_e4m3fn),
        pltpu.VMEM((_GR * L, D), jnp.float8_e4m3fn),
        pltpu.VMEM((_GR * L, D), jnp.float8_e4m3fn),
        pltpu.VMEM((_GR * L, D), jnp.float8_e4m3fn),
        pltpu.VMEM((RB // 8, 8, 2 * D), jnp.float32),
        pltpu.VMEM((_VCHUNK, D), jnp.float32),
        pltpu.VMEM((_VCHUNK, D), jnp.float8_e4m3fn),
        pltpu.VMEM_SHARED((VOCAB_PAD, D), jnp.float8_e4m3fn),
        pltpu.SemaphoreType.DMA,
        pltpu.SemaphoreType.DMA,
        pltpu.SemaphoreType.DMA,
        pltpu.SemaphoreType.DMA,
    ],
)(_pool_body)


def _mlp_body(h_ref, w1_ref, b1_ref, w2_ref, b2_ref, w3_ref, b3_ref, out_ref):
    h = h_ref[:].reshape(_MLP_BLOCK, 2 * D).astype(jnp.bfloat16)
    z1 = jnp.maximum(
        jnp.dot(h, w1_ref[:], preferred_element_type=jnp.float32) + b1_ref[:], 0.0
    ).astype(jnp.bfloat16)
    z2 = jnp.maximum(
        jnp.dot(z1, w2_ref[:], preferred_element_type=jnp.float32) + b2_ref[:], 0.0
    ).astype(jnp.bfloat16)
    logits = jnp.dot(z2, w3_ref[:], preferred_element_type=jnp.float32) + b3_ref[:]
    m = jnp.max(logits, axis=-1, keepdims=True)
    e = jnp.exp(logits - m)
    s = jnp.sum(e, axis=-1, keepdims=True)
    out_ref[:] = logits - m - jnp.log(s)


_MLP_BLOCK = 2048


def _mlp(h3, W1p, b1, W2, b2, W3, b3):
    grid = (B // _MLP_BLOCK,)
    full = lambda shape: pl.BlockSpec(shape, lambda i: tuple(0 for _ in shape))
    return pl.pallas_call(
        _mlp_body,
        grid=grid,
        in_specs=[
            pl.BlockSpec((_MLP_BLOCK // 8, 8, 2 * D), lambda i: (i, 0, 0)),
            full((2 * D, H)),
            full((1, H)),
            full((H, H)),
            full((1, H)),
            full((H, C)),
            full((1, C)),
        ],
        out_specs=pl.BlockSpec((_MLP_BLOCK, C), lambda i: (i, 0)),
        out_shape=jax.ShapeDtypeStruct((B, C), jnp.float32),
    )(h3, W1p, b1, W2, b2, W3, b3)


def kernel(x, emb, W1, b1, W2, b2, W3, b3):
    xf = x.astype(jnp.int32).reshape(-1)
    h3 = _pool(xf, emb)
    W1p = jnp.pad(W1, ((0, H - D), (0, 0))).astype(jnp.bfloat16)
    return _mlp(
        h3,
        W1p, b1.reshape(1, H),
        W2.astype(jnp.bfloat16), b2.reshape(1, H),
        W3.astype(jnp.bfloat16), b3.reshape(1, C),
    )


# speedup vs baseline: 71.0352x; 1.0378x over previous
import functools

import jax
import jax.numpy as jnp
from jax import lax
from jax.experimental import pallas as pl
from jax.experimental.pallas import tpu as pltpu
from jax.experimental.pallas import tpu_sc as plsc

VOCAB = 14923
D = 64
H = 128
C = 10
B = 16384
L = 200

NC, NS = 2, 16
NW = NC * NS
ROWS_PER_W = B // NW
RB = 128
N_CHUNKS = ROWS_PER_W // RB
NB = 4

_LANES = 16
_DV = D // _LANES
_TG = 25
_GR = 2
_SCALE = 64.0

_VSLICE = 933
_VCHUNK = 311
VOCAB_PAD = 16 * _VSLICE


def _pool_body(x_hbm, emb_hbm, out_hbm, idxs_v, r0_v, r1_v, r2_v, r3_v,
               out_v, stage_v, conv_v, spm, s0, s1, s2, s3):
    rows = (r0_v, r1_v, r2_v, r3_v)
    sems = (s0, s1, s2, s3)
    sid = lax.axis_index("s")
    wid = sid * NC + lax.axis_index("c")
    base = wid * ROWS_PER_W

    scale = jnp.float32(_SCALE)
    for c in range(3):
        src0 = sid * _VSLICE + c * _VCHUNK

        if c < 2:
            pltpu.sync_copy(emb_hbm.at[pl.ds(src0, _VCHUNK)], stage_v)
        else:
            @pl.when(sid < NS - 1)
            def _(src0=src0):
                pltpu.sync_copy(emb_hbm.at[pl.ds(src0, _VCHUNK)], stage_v)

            @pl.when(sid == NS - 1)
            def _(src0=src0):
                pltpu.sync_copy(
                    emb_hbm.at[pl.ds(src0, _VCHUNK - 5)],
                    stage_v.at[pl.ds(0, _VCHUNK - 5)],
                )

        def conv_body(i, _):
            w = stage_v[i, pl.ds(0, 16)] * scale
            x_ = stage_v[i, pl.ds(16, 16)] * scale
            y = stage_v[i, pl.ds(32, 16)] * scale
            z = stage_v[i, pl.ds(48, 16)] * scale
            a = plsc.pack(w, x_, format=plsc.PackFormat.INTERLEAVED)
            bvec = plsc.pack(y, z, format=plsc.PackFormat.INTERLEAVED)
            conv_v[i, pl.ds(0, 64)] = plsc.pack(
                a, bvec, format=plsc.PackFormat.INTERLEAVED,
                preferred_element_type=jnp.float8_e4m3fn,
            )
            return ()

        lax.fori_loop(0, _VCHUNK, conv_body, ())
        pltpu.sync_copy(conv_v, spm.at[pl.ds(src0, _VCHUNK)])

    plsc.subcore_barrier()

    inv_l = jnp.float32(1.0 / (L * _SCALE))

    zero16 = jnp.zeros((_LANES,), jnp.float32)

    def zinit_body(r, _):
        for j in range(4, 8):
            out_v[r // 8, r % 8, pl.ds(j * _LANES, _LANES)] = zero16
        return ()

    lax.fori_loop(0, RB, zinit_body, ())

    def chunk_body(c, _):
        row0 = base + c * RB
        pltpu.sync_copy(x_hbm.at[pl.ds(row0 * L, RB * L)], idxs_v)
        for b in range(NB):
            pltpu.async_copy(
                spm.at[idxs_v.at[pl.ds(b * _GR * L, _GR * L)]],
                rows[b], sems[b],
            )

        def group_body(g, _):
            for b in range(NB):
                p = g * NB + b
                pltpu.make_async_copy(
                    spm.at[idxs_v.at[pl.ds(0, _GR * L)]], rows[b], sems[b]
                ).wait()

                for rr in range(_GR):

                    def tok_body(ti, accs, b=b, rr=rr):
                        acc_a = jnp.zeros((32,), jnp.bfloat16)
                        acc_b = jnp.zeros((32,), jnp.bfloat16)
                        for u in range(_TG):
                            t = rr * L + ti * _TG + u
                            v = rows[b][t, pl.ds(0, 64)]
                            ea, eb = plsc.unpack(
                                v, format=plsc.PackFormat.INTERLEAVED,
                                preferred_element_type=jnp.bfloat16,
                            )
                            acc_a = acc_a + ea
                            acc_b = acc_b + eb
                        aa, ab = plsc.unpack(
                            acc_a, format=plsc.PackFormat.INTERLEAVED
                        )
                        ba, bb = plsc.unpack(
                            acc_b, format=plsc.PackFormat.INTERLEAVED
                        )
                        return (accs[0] + aa, accs[1] + ab,
                                accs[2] + ba, accs[3] + bb)

                    accs = lax.fori_loop(
                        0, L // _TG, tok_body,
                        tuple(jnp.zeros((_LANES,), jnp.float32)
                              for _ in range(_DV)),
                    )
                    r = p * _GR + rr
                    for j in range(_DV):
                        out_v[r // 8, r % 8, pl.ds(j * _LANES, _LANES)] = (
                            accs[j] * inv_l
                        )

                nxt = p + NB

                @pl.when(nxt < RB // _GR)
                def _(b=b, nxt=nxt):
                    pltpu.async_copy(
                        spm.at[idxs_v.at[pl.ds(nxt * _GR * L, _GR * L)]],
                        rows[b], sems[b],
                    )

            return ()

        lax.fori_loop(0, RB // _GR // NB, group_body, ())
        pltpu.sync_copy(out_v, out_hbm.at[pl.ds(row0 // 8, RB // 8)])
        return ()

    lax.fori_loop(0, N_CHUNKS, chunk_body, ())


_pool = functools.partial(
    pl.kernel,
    mesh=plsc.VectorSubcoreMesh(core_axis_name="c", subcore_axis_name="s"),
    out_type=jax.ShapeDtypeStruct((B // 8, 8, 2 * D), jnp.float32),
    compiler_params=pltpu.CompilerParams(
        use_tc_tiling_on_sc=False, needs_layout_passes=False
    ),
    scratch_types=[
        pltpu.VMEM((RB * L,), jnp.int32),
        pltpu.VMEM((_GR * L, D), jnp.float8_e4m3fn),
        pltpu.VMEM((_GR * L, D), jnp.float8_e4m3fn),
        pltpu.VMEM((_GR * L, D), jnp.float8_e4m3fn),
        pltpu.VMEM((_GR * L, D), jnp.float8_e4m3fn),
        pltpu.VMEM((RB // 8, 8, 2 * D), jnp.float32),
        pltpu.VMEM((_VCHUNK, D), jnp.float32),
        pltpu.VMEM((_VCHUNK, D), jnp.float8_e4m3fn),
        pltpu.VMEM_SHARED((VOCAB_PAD, D), jnp.float8_e4m3fn),
        pltpu.SemaphoreType.DMA,
        pltpu.SemaphoreType.DMA,
        pltpu.SemaphoreType.DMA,
        pltpu.SemaphoreType.DMA,
    ],
)(_pool_body)


def _mlp_body(h_ref, w1_ref, b1_ref, w2_ref, b2_ref, w3_ref, b3_ref, out_ref):
    h = h_ref[:].reshape(_MLP_BLOCK, 2 * D).astype(jnp.bfloat16)
    z1 = jnp.maximum(
        jnp.dot(h, w1_ref[:], preferred_element_type=jnp.float32) + b1_ref[:], 0.0
    ).astype(jnp.bfloat16)
    z2 = jnp.maximum(
        jnp.dot(z1, w2_ref[:], preferred_element_type=jnp.float32) + b2_ref[:], 0.0
    ).astype(jnp.bfloat16)
    logits = jnp.dot(z2, w3_ref[:], preferred_element_type=jnp.float32) + b3_ref[:]
    m = jnp.max(logits, axis=-1, keepdims=True)
    e = jnp.exp(logits - m)
    s = jnp.sum(e, axis=-1, keepdims=True)
    out_ref[:] = logits - m - jnp.log(s)


_MLP_BLOCK = 4096


def _mlp(h3, W1p, b1, W2, b2, W3, b3):
    grid = (B // _MLP_BLOCK,)
    full = lambda shape: pl.BlockSpec(shape, lambda i: tuple(0 for _ in shape))
    return pl.pallas_call(
        _mlp_body,
        grid=grid,
        in_specs=[
            pl.BlockSpec((_MLP_BLOCK // 8, 8, 2 * D), lambda i: (i, 0, 0)),
            full((2 * D, H)),
            full((1, H)),
            full((H, H)),
            full((1, H)),
            full((H, C)),
            full((1, C)),
        ],
        out_specs=pl.BlockSpec((_MLP_BLOCK, C), lambda i: (i, 0)),
        out_shape=jax.ShapeDtypeStruct((B, C), jnp.float32),
    )(h3, W1p, b1, W2, b2, W3, b3)


def kernel(x, emb, W1, b1, W2, b2, W3, b3):
    xf = x.astype(jnp.int32).reshape(-1)
    h3 = _pool(xf, emb)
    W1p = jnp.pad(W1, ((0, H - D), (0, 0))).astype(jnp.bfloat16)
    return _mlp(
        h3,
        W1p, b1.reshape(1, H),
        W2.astype(jnp.bfloat16), b2.reshape(1, H),
        W3.astype(jnp.bfloat16), b3.reshape(1, C),
    )
